# baseline (speedup 1.0000x reference)
import jax
import jax.numpy as jnp
from jax import lax
from jax.experimental import pallas as pl

N = 128
TOT = 4096 * 200
BLK = 2048
NB = TOT // BLK


def _body(z_ref, o_ref):
    z = z_ref[0, 0, :].reshape(BLK, 1)
    col = lax.broadcasted_iota(jnp.int32, (BLK, N), 1)
    o_ref[0] = (z == col).astype(jnp.float32)


def kernel(Z, eye):
    del eye
    zr = Z.reshape(NB, 1, BLK).astype(jnp.int32)
    out = pl.pallas_call(
        _body,
        grid=(NB,),
        in_specs=[pl.BlockSpec((1, 1, BLK), lambda i: (i, 0, 0))],
        out_specs=pl.BlockSpec((1, BLK, N), lambda i: (i, 0, 0)),
        out_shape=jax.ShapeDtypeStruct((NB, BLK, N), jnp.float32),
    )(zr)
    return out.reshape(Z.shape + (N,))


# speedup vs baseline: 1.7564x; 1.7564x over previous
import functools

import jax
import jax.numpy as jnp
from jax import lax
from jax.experimental import pallas as pl
from jax.experimental.pallas import tpu as pltpu
from jax.experimental.pallas import tpu_sc as plsc

N = 128
NC, NS = 2, 16
NW = NC * NS
TOT = 4096 * 200
CPW = TOT // NW
C = 400
NCHUNK = CPW // C
NPAIR = NCHUNK // 2

_mesh = plsc.VectorSubcoreMesh(core_axis_name="c", subcore_axis_name="s")


@functools.partial(
    pl.kernel,
    mesh=_mesh,
    out_type=jax.ShapeDtypeStruct((TOT, N), jnp.float32),
    scratch_types=[
        pltpu.VMEM((CPW,), jnp.int32),
        pltpu.VMEM((C, N), jnp.float32),
        pltpu.VMEM((C, N), jnp.float32),
        pltpu.SemaphoreType.DMA,
        pltpu.SemaphoreType.DMA,
    ],
    compiler_params=pltpu.CompilerParams(needs_layout_passes=False),
)
def _one_hot_sc(idx_hbm, zeros_hbm, out_hbm, idx_v, rows0, rows1, sem0, sem1):
    wid = lax.axis_index("s") * NC + lax.axis_index("c")
    lane = lax.iota(jnp.int32, 16)
    ones = jnp.full((16,), 1.0, jnp.float32)
    zeros = jnp.zeros((16,), jnp.float32)
    wbase = wid * CPW

    pltpu.sync_copy(idx_hbm.at[pl.ds(wbase, CPW)], idx_v)
    pltpu.sync_copy(zeros_hbm, rows0)
    pltpu.sync_copy(zeros_hbm, rows1)

    def scatter(buf, c, val):
        for i in range(C // 16):
            rows = lane + i * 16
            cols = idx_v[pl.ds(c * C + i * 16, 16)]
            plsc.store_scatter(buf, [rows, cols], val)

    def pair(p, carry):
        for q, (buf, sem) in enumerate(((rows0, sem0), (rows1, sem1))):
            c = p * 2 + q

            @pl.when(p > 0)
            def _recycle():
                pltpu.make_async_copy(
                    buf, out_hbm.at[pl.ds(wbase, C)], sem).wait()
                scatter(buf, c - 2, zeros)

            scatter(buf, c, ones)
            pltpu.async_copy(buf, out_hbm.at[pl.ds(wbase + c * C, C)], sem)
        return carry

    lax.fori_loop(0, NPAIR, pair, 0)
    for buf, sem in ((rows0, sem0), (rows1, sem1)):
        pltpu.make_async_copy(buf, out_hbm.at[pl.ds(wbase, C)], sem).wait()


def kernel(Z, eye):
    del eye
    idx = Z.reshape(-1).astype(jnp.int32)
    zeros = jnp.zeros((C, N), jnp.float32)
    out = _one_hot_sc(idx, zeros)
    return out.reshape(Z.shape + (N,))


# speedup vs baseline: 1.7862x; 1.0170x over previous
import functools

import jax
import jax.numpy as jnp
from jax import lax
from jax.experimental import pallas as pl
from jax.experimental.pallas import tpu as pltpu
from jax.experimental.pallas import tpu_sc as plsc

N = 128
NC, NS = 2, 16
NW = NC * NS
TOT = 4096 * 200
CPW = TOT // NW
C = 160
NBUF = 4
NCHUNK = CPW // C
NGRP = NCHUNK // NBUF

_mesh = plsc.VectorSubcoreMesh(core_axis_name="c", subcore_axis_name="s")


@functools.partial(
    pl.kernel,
    mesh=_mesh,
    out_type=jax.ShapeDtypeStruct((TOT, N), jnp.float32),
    scratch_types=[
        pltpu.VMEM((CPW,), jnp.int32),
        pltpu.VMEM((C, N), jnp.float32),
        pltpu.VMEM((C, N), jnp.float32),
        pltpu.VMEM((C, N), jnp.float32),
        pltpu.VMEM((C, N), jnp.float32),
        pltpu.SemaphoreType.DMA,
        pltpu.SemaphoreType.DMA,
        pltpu.SemaphoreType.DMA,
        pltpu.SemaphoreType.DMA,
    ],
    compiler_params=pltpu.CompilerParams(needs_layout_passes=False),
)
def _one_hot_sc(idx_hbm, zeros_hbm, out_hbm, idx_v,
                rows0, rows1, rows2, rows3, sem0, sem1, sem2, sem3):
    wid = lax.axis_index("s") * NC + lax.axis_index("c")
    lane = lax.iota(jnp.int32, 16)
    ones = jnp.full((16,), 1.0, jnp.float32)
    zeros = jnp.zeros((16,), jnp.float32)
    wbase = wid * CPW

    bufs = ((rows0, sem0), (rows1, sem1), (rows2, sem2), (rows3, sem3))

    pltpu.sync_copy(idx_hbm.at[pl.ds(wbase, CPW)], idx_v)
    for buf, _ in bufs:
        pltpu.sync_copy(zeros_hbm, buf)

    def scatter(buf, c, val):
        for i in range(C // 16):
            rows = lane + i * 16
            cols = idx_v[pl.ds(c * C + i * 16, 16)]
            plsc.store_scatter(buf, [rows, cols], val)

    def group(p, carry):
        for q, (buf, sem) in enumerate(bufs):
            c = p * NBUF + q

            @pl.when(p > 0)
            def _recycle():
                pltpu.make_async_copy(
                    buf, out_hbm.at[pl.ds(wbase, C)], sem).wait()
                scatter(buf, c - NBUF, zeros)

            scatter(buf, c, ones)
            pltpu.async_copy(buf, out_hbm.at[pl.ds(wbase + c * C, C)], sem)
        return carry

    lax.fori_loop(0, NGRP, group, 0)
    for buf, sem in bufs:
        pltpu.make_async_copy(buf, out_hbm.at[pl.ds(wbase, C)], sem).wait()


def kernel(Z, eye):
    del eye
    idx = Z.reshape(-1).astype(jnp.int32)
    zeros = jnp.zeros((C, N), jnp.float32)
    out = _one_hot_sc(idx, zeros)
    return out.reshape(Z.shape + (N,))
